# Initial kernel scaffold; baseline (speedup 1.0000x reference)
import jax
import jax.numpy as jnp
from jax.experimental import pallas as pl


def kernel(edge_weight, edge_index):
    raise NotImplementedError("write your pallas kernel here")


# speedup vs baseline: 123.6879x; 123.6879x over previous
import functools

import jax
import jax.numpy as jnp
from jax import lax
from jax.experimental import pallas as pl
from jax.experimental.pallas import tpu as pltpu
from jax.experimental.pallas import tpu_sc as plsc

N_NODES = 100000
N_EDGES = 6400000

NC = 2
NS = 16
NW = NC * NS

L = 16
CHUNK = 2000

NPAD = 100096
NODES_PER_TILE = NPAD // NS

EDGES_PER_TILE1 = N_EDGES // NS
ITERS1 = EDGES_PER_TILE1 // CHUNK
EDGES_PER_W3 = N_EDGES // NW
ITERS3 = EDGES_PER_W3 // CHUNK


def _rsqrt16(x):
    i = lax.bitcast_convert_type(x, jnp.int32)
    i = jnp.int32(0x5F3759DF) - (i >> 1)
    y = lax.bitcast_convert_type(i, jnp.float32)
    half = x * jnp.float32(0.5)
    for _ in range(3):
        y = y * (jnp.float32(1.5) - half * y * y)
    return y


def _body(src_hbm, dst_hbm, w_hbm, out_hbm,
          sidx_v, didx_v, w_v, a_v, b_v, o_v, zbuf_v, nbuf_v,
          deg_src_sh, deg_dst_sh):
    c = lax.axis_index("c")
    s = lax.axis_index("s")
    wid = s * NC + c

    def zero_body(j, _):
        zbuf_v[pl.ds(j * L, L)] = jnp.zeros((L,), jnp.float32)
        return _
    lax.fori_loop(0, NODES_PER_TILE // L, zero_body, None)
    node_base = s * NODES_PER_TILE
    pltpu.sync_copy(zbuf_v, deg_src_sh.at[pl.ds(node_base, NODES_PER_TILE)])
    pltpu.sync_copy(zbuf_v, deg_dst_sh.at[pl.ds(node_base, NODES_PER_TILE)])
    plsc.subcore_barrier()

    base1 = s * EDGES_PER_TILE1

    def p1_body(j, _):
        e = base1 + j * CHUNK
        pltpu.sync_copy(w_hbm.at[pl.ds(e, CHUNK)], w_v)
        pltpu.sync_copy(src_hbm.at[pl.ds(e, CHUNK)], sidx_v)
        pltpu.sync_copy(dst_hbm.at[pl.ds(e, CHUNK)], didx_v)
        pltpu.sync_copy(w_v, deg_src_sh.at[sidx_v], add=True)
        pltpu.sync_copy(w_v, deg_dst_sh.at[didx_v], add=True)
        return _
    lax.fori_loop(0, ITERS1, p1_body, None)
    plsc.subcore_barrier()

    def norm_table(table):
        pltpu.sync_copy(table.at[pl.ds(node_base, NODES_PER_TILE)], zbuf_v)

        def n_body(j, _):
            x = zbuf_v[pl.ds(j * L, L)]
            nbuf_v[pl.ds(j * L, L)] = _rsqrt16(x)
            return _
        lax.fori_loop(0, NODES_PER_TILE // L, n_body, None)
        pltpu.sync_copy(nbuf_v, table.at[pl.ds(node_base, NODES_PER_TILE)])
    norm_table(deg_src_sh)
    norm_table(deg_dst_sh)
    plsc.subcore_barrier()

    base3 = wid * EDGES_PER_W3

    def p3_body(j, _):
        e = base3 + j * CHUNK
        pltpu.sync_copy(src_hbm.at[pl.ds(e, CHUNK)], sidx_v)
        pltpu.sync_copy(dst_hbm.at[pl.ds(e, CHUNK)], didx_v)
        pltpu.sync_copy(w_hbm.at[pl.ds(e, CHUNK)], w_v)
        pltpu.sync_copy(deg_src_sh.at[sidx_v], a_v)
        pltpu.sync_copy(deg_dst_sh.at[didx_v], b_v)

        def mul_body(i, _):
            sl = pl.ds(i * L, L)
            o_v[sl] = a_v[sl] * b_v[sl] * w_v[sl]
            return _
        lax.fori_loop(0, CHUNK // L, mul_body, None)
        pltpu.sync_copy(o_v, out_hbm.at[pl.ds(e, CHUNK)])
        return _
    lax.fori_loop(0, ITERS3, p3_body, None)


@jax.jit
def kernel(edge_weight, edge_index):
    src = edge_index[0].astype(jnp.int32)
    dst = edge_index[1].astype(jnp.int32)

    mesh = plsc.VectorSubcoreMesh(core_axis_name="c", subcore_axis_name="s")
    chunk_i32 = pltpu.VMEM((CHUNK,), jnp.int32)
    chunk_f32 = pltpu.VMEM((CHUNK,), jnp.float32)
    kern = pl.kernel(
        _body,
        out_type=jax.ShapeDtypeStruct((N_EDGES,), jnp.float32),
        mesh=mesh,
        scratch_types=[
            chunk_i32, chunk_i32,
            chunk_f32, chunk_f32, chunk_f32, chunk_f32,
            pltpu.VMEM((NODES_PER_TILE,), jnp.float32),
            pltpu.VMEM((NODES_PER_TILE,), jnp.float32),
            pltpu.VMEM_SHARED((NPAD,), jnp.float32),
            pltpu.VMEM_SHARED((NPAD,), jnp.float32),
        ],
    )
    return kern(src, dst, edge_weight)


# speedup vs baseline: 256.3087x; 2.0722x over previous
import jax
import jax.numpy as jnp
from jax import lax
from jax.experimental import pallas as pl
from jax.experimental.pallas import tpu as pltpu
from jax.experimental.pallas import tpu_sc as plsc

N_NODES = 100000
N_EDGES = 6400000

NC = 2
NS = 16
NW = NC * NS

L = 16
CHUNK = 8000

NPAD = 100096
NPT = NPAD // NS

EDGES_PER_W = N_EDGES // NW
ITERS = EDGES_PER_W // CHUNK


def _rsqrt16(x):
    i = lax.bitcast_convert_type(x, jnp.int32)
    i = jnp.int32(0x5F3759DF) - (i >> 1)
    y = lax.bitcast_convert_type(i, jnp.float32)
    half = x * jnp.float32(0.5)
    for _ in range(3):
        y = y * (jnp.float32(1.5) - half * y * y)
    return y


def _body_a(src_hbm, dst_hbm, w_hbm, p_hbm,
            sidx_v, didx_v, w_v, zbuf_v, deg_src_sh, deg_dst_sh):
    c = lax.axis_index("c")
    s = lax.axis_index("s")

    def zero_body(j, _):
        zbuf_v[pl.ds(j * L, L)] = jnp.zeros((L,), jnp.float32)
        return _
    lax.fori_loop(0, NPT // L, zero_body, None)
    node_base = s * NPT
    pltpu.sync_copy(zbuf_v, deg_src_sh.at[pl.ds(node_base, NPT)])
    pltpu.sync_copy(zbuf_v, deg_dst_sh.at[pl.ds(node_base, NPT)])
    plsc.subcore_barrier()

    base = (c * NS + s) * EDGES_PER_W

    def p1_body(j, _):
        e = base + j * CHUNK
        pltpu.sync_copy(w_hbm.at[pl.ds(e, CHUNK)], w_v)
        pltpu.sync_copy(src_hbm.at[pl.ds(e, CHUNK)], sidx_v)
        pltpu.sync_copy(dst_hbm.at[pl.ds(e, CHUNK)], didx_v)
        pltpu.sync_copy(w_v, deg_src_sh.at[sidx_v], add=True)
        pltpu.sync_copy(w_v, deg_dst_sh.at[didx_v], add=True)
        return _
    lax.fori_loop(0, ITERS, p1_body, None)
    plsc.subcore_barrier()

    pltpu.sync_copy(deg_src_sh.at[pl.ds(node_base, NPT)], zbuf_v)
    pltpu.sync_copy(zbuf_v,
                    p_hbm.at[pl.ds((c * 2 + 0) * NPAD + node_base, NPT)])
    pltpu.sync_copy(deg_dst_sh.at[pl.ds(node_base, NPT)], zbuf_v)
    pltpu.sync_copy(zbuf_v,
                    p_hbm.at[pl.ds((c * 2 + 1) * NPAD + node_base, NPT)])


def _body_b(src_hbm, dst_hbm, w_hbm, p_hbm, out_hbm,
            sidx_v, didx_v, w_v, a_v, b_v, o_v, pbuf0_v, pbuf1_v,
            norm_src_sh, norm_dst_sh):
    c = lax.axis_index("c")
    s = lax.axis_index("s")
    wid = c * NS + s
    node_base = s * NPT

    def build(t, table_sh):
        pltpu.sync_copy(p_hbm.at[pl.ds(t * NPAD + node_base, NPT)], pbuf0_v)
        pltpu.sync_copy(p_hbm.at[pl.ds((2 + t) * NPAD + node_base, NPT)],
                        pbuf1_v)

        def n_body(j, _):
            sl = pl.ds(j * L, L)
            pbuf0_v[sl] = _rsqrt16(pbuf0_v[sl] + pbuf1_v[sl])
            return _
        lax.fori_loop(0, NPT // L, n_body, None)
        pltpu.sync_copy(pbuf0_v, table_sh.at[pl.ds(node_base, NPT)])
    build(0, norm_src_sh)
    build(1, norm_dst_sh)
    plsc.subcore_barrier()

    base = wid * EDGES_PER_W

    def p3_body(j, _):
        e = base + j * CHUNK
        pltpu.sync_copy(src_hbm.at[pl.ds(e, CHUNK)], sidx_v)
        pltpu.sync_copy(dst_hbm.at[pl.ds(e, CHUNK)], didx_v)
        pltpu.sync_copy(w_hbm.at[pl.ds(e, CHUNK)], w_v)
        pltpu.sync_copy(norm_src_sh.at[sidx_v], a_v)
        pltpu.sync_copy(norm_dst_sh.at[didx_v], b_v)

        def mul_body(i, _):
            sl = pl.ds(i * L, L)
            o_v[sl] = a_v[sl] * b_v[sl] * w_v[sl]
            return _
        lax.fori_loop(0, CHUNK // L, mul_body, None)
        pltpu.sync_copy(o_v, out_hbm.at[pl.ds(e, CHUNK)])
        return _
    lax.fori_loop(0, ITERS, p3_body, None)


@jax.jit
def kernel(edge_weight, edge_index):
    src = edge_index[0].astype(jnp.int32)
    dst = edge_index[1].astype(jnp.int32)

    mesh = plsc.VectorSubcoreMesh(core_axis_name="c", subcore_axis_name="s")
    chunk_i32 = pltpu.VMEM((CHUNK,), jnp.int32)
    chunk_f32 = pltpu.VMEM((CHUNK,), jnp.float32)
    npt_f32 = pltpu.VMEM((NPT,), jnp.float32)
    shared_f32 = pltpu.VMEM_SHARED((NPAD,), jnp.float32)

    kern_a = pl.kernel(
        _body_a,
        out_type=jax.ShapeDtypeStruct((NC * 2 * NPAD,), jnp.float32),
        mesh=mesh,
        scratch_types=[chunk_i32, chunk_i32, chunk_f32, npt_f32,
                       shared_f32, shared_f32],
    )
    kern_b = pl.kernel(
        _body_b,
        out_type=jax.ShapeDtypeStruct((N_EDGES,), jnp.float32),
        mesh=mesh,
        scratch_types=[chunk_i32, chunk_i32,
                       chunk_f32, chunk_f32, chunk_f32, chunk_f32,
                       npt_f32, npt_f32,
                       shared_f32, shared_f32],
    )
    partials = kern_a(src, dst, edge_weight)
    return kern_b(src, dst, edge_weight, partials)


# speedup vs baseline: 357.3609x; 1.3943x over previous
import jax
import jax.numpy as jnp
from jax import lax
from jax.experimental import pallas as pl
from jax.experimental.pallas import tpu as pltpu
from jax.experimental.pallas import tpu_sc as plsc

N_NODES = 100000
N_EDGES = 6400000

NC = 2
NS = 16
NW = NC * NS

L = 16

NPAD = 100096
NPT = NPAD // NS

EDGES_PER_W = N_EDGES // NW
CHUNK_A = 10000
ITERS_A = EDGES_PER_W // CHUNK_A
CHUNK_B = 4000
ITERS_B = EDGES_PER_W // CHUNK_B


def _rsqrt16(x):
    i = lax.bitcast_convert_type(x, jnp.int32)
    i = jnp.int32(0x5F3759DF) - (i >> 1)
    y = lax.bitcast_convert_type(i, jnp.float32)
    half = x * jnp.float32(0.5)
    for _ in range(3):
        y = y * (jnp.float32(1.5) - half * y * y)
    return y


def _body_a(src_hbm, dst_hbm, w_hbm, p_hbm,
            sidx0_v, sidx1_v, didx0_v, didx1_v, w0_v, w1_v,
            zbuf_v, lsem, ssem, deg_src_sh, deg_dst_sh):
    sidx_v = (sidx0_v, sidx1_v)
    didx_v = (didx0_v, didx1_v)
    w_v = (w0_v, w1_v)
    c = lax.axis_index("c")
    s = lax.axis_index("s")
    node_base = s * NPT
    base = (c * NS + s) * EDGES_PER_W

    def load(b, j):
        e = base + j * CHUNK_A
        pltpu.async_copy(w_hbm.at[pl.ds(e, CHUNK_A)], w_v[b], lsem.at[b])
        pltpu.async_copy(src_hbm.at[pl.ds(e, CHUNK_A)], sidx_v[b],
                         lsem.at[b])
        pltpu.async_copy(dst_hbm.at[pl.ds(e, CHUNK_A)], didx_v[b],
                         lsem.at[b])

    def wait_load(b, j):
        e = base + j * CHUNK_A
        pltpu.make_async_copy(w_hbm.at[pl.ds(e, CHUNK_A)], w_v[b],
                              lsem.at[b]).wait()
        pltpu.make_async_copy(src_hbm.at[pl.ds(e, CHUNK_A)], sidx_v[b],
                              lsem.at[b]).wait()
        pltpu.make_async_copy(dst_hbm.at[pl.ds(e, CHUNK_A)], didx_v[b],
                              lsem.at[b]).wait()

    def scatter(b):
        pltpu.async_copy(w_v[b], deg_src_sh.at[sidx_v[b]], ssem.at[b],
                         add=True)
        pltpu.async_copy(w_v[b], deg_dst_sh.at[didx_v[b]], ssem.at[b],
                         add=True)

    def wait_scatter(b):
        pltpu.make_async_copy(w_v[b], deg_src_sh.at[sidx_v[b]],
                              ssem.at[b]).wait()
        pltpu.make_async_copy(w_v[b], deg_dst_sh.at[didx_v[b]],
                              ssem.at[b]).wait()

    load(0, 0)

    def zero_body(j, _):
        zbuf_v[pl.ds(j * L, L)] = jnp.zeros((L,), jnp.float32)
        return _
    lax.fori_loop(0, NPT // L, zero_body, None)
    pltpu.sync_copy(zbuf_v, deg_src_sh.at[pl.ds(node_base, NPT)])
    pltpu.sync_copy(zbuf_v, deg_dst_sh.at[pl.ds(node_base, NPT)])
    plsc.subcore_barrier()

    wait_load(0, 0)
    scatter(0)
    load(1, 1)

    def ring(g, _):
        for b, off in ((1, 1), (0, 2)):
            j = 2 * g + off
            wait_load(b, j)
            scatter(b)
            wait_scatter(1 - b)
            load(1 - b, j + 1)
        return _
    lax.fori_loop(0, (ITERS_A - 2) // 2, ring, None)

    wait_load(1, ITERS_A - 1)
    scatter(1)
    wait_scatter(0)
    wait_scatter(1)
    plsc.subcore_barrier()

    pltpu.sync_copy(deg_src_sh.at[pl.ds(node_base, NPT)], zbuf_v)
    pltpu.sync_copy(zbuf_v,
                    p_hbm.at[pl.ds((c * 2 + 0) * NPAD + node_base, NPT)])
    pltpu.sync_copy(deg_dst_sh.at[pl.ds(node_base, NPT)], zbuf_v)
    pltpu.sync_copy(zbuf_v,
                    p_hbm.at[pl.ds((c * 2 + 1) * NPAD + node_base, NPT)])


def _body_b(src_hbm, dst_hbm, w_hbm, p_hbm, out_hbm,
            sidx0_v, sidx1_v, didx0_v, didx1_v, w0_v, w1_v,
            a0_v, a1_v, b0_v, b1_v, o0_v, o1_v, pbuf0_v, pbuf1_v,
            lsem, gsem, osem, norm_src_sh, norm_dst_sh):
    sidx_v = (sidx0_v, sidx1_v)
    didx_v = (didx0_v, didx1_v)
    w_v = (w0_v, w1_v)
    a_v = (a0_v, a1_v)
    b_v = (b0_v, b1_v)
    o_v = (o0_v, o1_v)
    c = lax.axis_index("c")
    s = lax.axis_index("s")
    wid = c * NS + s
    node_base = s * NPT
    base = wid * EDGES_PER_W

    def load(b, j):
        e = base + j * CHUNK_B
        pltpu.async_copy(src_hbm.at[pl.ds(e, CHUNK_B)], sidx_v[b],
                         lsem.at[b])
        pltpu.async_copy(dst_hbm.at[pl.ds(e, CHUNK_B)], didx_v[b],
                         lsem.at[b])
        pltpu.async_copy(w_hbm.at[pl.ds(e, CHUNK_B)], w_v[b], lsem.at[b])

    def wait_load(b, j):
        e = base + j * CHUNK_B
        pltpu.make_async_copy(src_hbm.at[pl.ds(e, CHUNK_B)], sidx_v[b],
                              lsem.at[b]).wait()
        pltpu.make_async_copy(dst_hbm.at[pl.ds(e, CHUNK_B)], didx_v[b],
                              lsem.at[b]).wait()
        pltpu.make_async_copy(w_hbm.at[pl.ds(e, CHUNK_B)], w_v[b],
                              lsem.at[b]).wait()

    def gather(b):
        pltpu.async_copy(norm_src_sh.at[sidx_v[b]], a_v[b], gsem.at[b])
        pltpu.async_copy(norm_dst_sh.at[didx_v[b]], b_v[b], gsem.at[b])

    def wait_gather(b):
        pltpu.make_async_copy(norm_src_sh.at[sidx_v[b]], a_v[b],
                              gsem.at[b]).wait()
        pltpu.make_async_copy(norm_dst_sh.at[didx_v[b]], b_v[b],
                              gsem.at[b]).wait()

    def mul(b):
        def mul_body(i, _):
            sl = pl.ds(i * L, L)
            o_v[b][sl] = a_v[b][sl] * b_v[b][sl] * w_v[b][sl]
            return _
        lax.fori_loop(0, CHUNK_B // L, mul_body, None)

    def store(b, j):
        e = base + j * CHUNK_B
        pltpu.async_copy(o_v[b], out_hbm.at[pl.ds(e, CHUNK_B)], osem.at[b])

    def wait_store(b, j):
        e = base + j * CHUNK_B
        pltpu.make_async_copy(o_v[b], out_hbm.at[pl.ds(e, CHUNK_B)],
                              osem.at[b]).wait()

    load(0, 0)
    load(1, 1)

    def build(t, table_sh):
        pltpu.sync_copy(p_hbm.at[pl.ds(t * NPAD + node_base, NPT)], pbuf0_v)
        pltpu.sync_copy(p_hbm.at[pl.ds((2 + t) * NPAD + node_base, NPT)],
                        pbuf1_v)

        def n_body(j, _):
            sl = pl.ds(j * L, L)
            pbuf0_v[sl] = _rsqrt16(pbuf0_v[sl] + pbuf1_v[sl])
            return _
        lax.fori_loop(0, NPT // L, n_body, None)
        pltpu.sync_copy(pbuf0_v, table_sh.at[pl.ds(node_base, NPT)])
    build(0, norm_src_sh)
    build(1, norm_dst_sh)
    plsc.subcore_barrier()

    def step(b, j, *, first=False, last=False):
        wait_load(b, j)
        gather(b)
        wait_gather(b)
        if not first:
            wait_store(b, j - 2)
        mul(b)
        store(b, j)
        if not last:
            load(b, j + 2)

    step(0, 0, first=True)
    step(1, 1, first=True)

    def ring(g, _):
        for b, off in ((0, 2), (1, 3)):
            step(b, 2 * g + off)
        return _
    lax.fori_loop(0, (ITERS_B - 4) // 2, ring, None)

    step(0, ITERS_B - 2, last=True)
    step(1, ITERS_B - 1, last=True)
    wait_store(0, ITERS_B - 2)
    wait_store(1, ITERS_B - 1)


@jax.jit
def kernel(edge_weight, edge_index):
    src = edge_index[0].astype(jnp.int32)
    dst = edge_index[1].astype(jnp.int32)

    mesh = plsc.VectorSubcoreMesh(core_axis_name="c", subcore_axis_name="s")
    npt_f32 = pltpu.VMEM((NPT,), jnp.float32)
    shared_f32 = pltpu.VMEM_SHARED((NPAD,), jnp.float32)
    sem2 = pltpu.SemaphoreType.DMA((2,))

    kern_a = pl.kernel(
        _body_a,
        out_type=jax.ShapeDtypeStruct((NC * 2 * NPAD,), jnp.float32),
        mesh=mesh,
        scratch_types=(
            [pltpu.VMEM((CHUNK_A,), jnp.int32)] * 4
            + [pltpu.VMEM((CHUNK_A,), jnp.float32)] * 2
            + [npt_f32, sem2, sem2, shared_f32, shared_f32]
        ),
    )
    kern_b = pl.kernel(
        _body_b,
        out_type=jax.ShapeDtypeStruct((N_EDGES,), jnp.float32),
        mesh=mesh,
        scratch_types=(
            [pltpu.VMEM((CHUNK_B,), jnp.int32)] * 4
            + [pltpu.VMEM((CHUNK_B,), jnp.float32)] * 8
            + [npt_f32, npt_f32, sem2, sem2, sem2, shared_f32, shared_f32]
        ),
    )
    partials = kern_a(src, dst, edge_weight)
    return kern_b(src, dst, edge_weight, partials)


# speedup vs baseline: 421.9714x; 1.1808x over previous
import jax
import jax.numpy as jnp
from jax import lax
from jax.experimental import pallas as pl
from jax.experimental.pallas import tpu as pltpu
from jax.experimental.pallas import tpu_sc as plsc

N_NODES = 100000
N_EDGES = 6400000

NC = 2
NS = 16
NW = NC * NS

L = 16

NPAD = 100096
NPT = NPAD // NS

EDGES_PER_W = N_EDGES // NW
CHUNK_A = 10000
ITERS_A = EDGES_PER_W // CHUNK_A
CHUNK_B = 8000
ITERS_B = EDGES_PER_W // CHUNK_B


def _rsqrt16(x):
    i = lax.bitcast_convert_type(x, jnp.int32)
    i = jnp.int32(0x5F3759DF) - (i >> 1)
    y = lax.bitcast_convert_type(i, jnp.float32)
    half = x * jnp.float32(0.5)
    for _ in range(3):
        y = y * (jnp.float32(1.5) - half * y * y)
    return y


def _body_a(eidx_hbm, w_hbm, p_hbm,
            sidx0_v, sidx1_v, didx0_v, didx1_v, w0_v, w1_v,
            zbuf_v, lsem, ssem, deg_src_sh, deg_dst_sh):
    sidx_v = (sidx0_v, sidx1_v)
    didx_v = (didx0_v, didx1_v)
    w_v = (w0_v, w1_v)
    c = lax.axis_index("c")
    s = lax.axis_index("s")
    node_base = s * NPT
    base = (c * NS + s) * EDGES_PER_W

    def load(b, j):
        e = base + j * CHUNK_A
        pltpu.async_copy(w_hbm.at[pl.ds(e, CHUNK_A)], w_v[b], lsem.at[b])
        pltpu.async_copy(eidx_hbm.at[pl.ds(e, CHUNK_A)], sidx_v[b],
                         lsem.at[b])
        pltpu.async_copy(eidx_hbm.at[pl.ds(N_EDGES + e, CHUNK_A)], didx_v[b],
                         lsem.at[b])

    def wait_load(b, j):
        e = base + j * CHUNK_A
        pltpu.make_async_copy(w_hbm.at[pl.ds(e, CHUNK_A)], w_v[b],
                              lsem.at[b]).wait()
        pltpu.make_async_copy(eidx_hbm.at[pl.ds(e, CHUNK_A)], sidx_v[b],
                              lsem.at[b]).wait()
        pltpu.make_async_copy(eidx_hbm.at[pl.ds(N_EDGES + e, CHUNK_A)],
                              didx_v[b], lsem.at[b]).wait()

    def scatter(b):
        pltpu.async_copy(w_v[b], deg_src_sh.at[sidx_v[b]], ssem.at[b],
                         add=True)
        pltpu.async_copy(w_v[b], deg_dst_sh.at[didx_v[b]], ssem.at[b],
                         add=True)

    def wait_scatter(b):
        pltpu.make_async_copy(w_v[b], deg_src_sh.at[sidx_v[b]],
                              ssem.at[b]).wait()
        pltpu.make_async_copy(w_v[b], deg_dst_sh.at[didx_v[b]],
                              ssem.at[b]).wait()

    load(0, 0)

    def zero_body(j, _):
        zbuf_v[pl.ds(j * L, L)] = jnp.zeros((L,), jnp.float32)
        return _
    lax.fori_loop(0, NPT // L, zero_body, None)
    pltpu.sync_copy(zbuf_v, deg_src_sh.at[pl.ds(node_base, NPT)])
    pltpu.sync_copy(zbuf_v, deg_dst_sh.at[pl.ds(node_base, NPT)])
    plsc.subcore_barrier()

    wait_load(0, 0)
    scatter(0)
    load(1, 1)

    def ring(g, _):
        for b, off in ((1, 1), (0, 2)):
            j = 2 * g + off
            wait_load(b, j)
            scatter(b)
            wait_scatter(1 - b)
            load(1 - b, j + 1)
        return _
    lax.fori_loop(0, (ITERS_A - 2) // 2, ring, None)

    wait_load(1, ITERS_A - 1)
    scatter(1)
    wait_scatter(0)
    wait_scatter(1)
    plsc.subcore_barrier()

    pltpu.sync_copy(deg_src_sh.at[pl.ds(node_base, NPT)], zbuf_v)
    pltpu.sync_copy(zbuf_v,
                    p_hbm.at[pl.ds((c * 2 + 0) * NPAD + node_base, NPT)])
    pltpu.sync_copy(deg_dst_sh.at[pl.ds(node_base, NPT)], zbuf_v)
    pltpu.sync_copy(zbuf_v,
                    p_hbm.at[pl.ds((c * 2 + 1) * NPAD + node_base, NPT)])


def _body_b(eidx_hbm, w_hbm, p_hbm, out_hbm,
            sidx0_v, sidx1_v, didx0_v, didx1_v, w0_v, w1_v,
            a0_v, a1_v, b0_v, b1_v, o0_v, o1_v, pbuf0_v, pbuf1_v,
            lsem, gsem, osem, norm_src_sh, norm_dst_sh):
    sidx_v = (sidx0_v, sidx1_v)
    didx_v = (didx0_v, didx1_v)
    w_v = (w0_v, w1_v)
    a_v = (a0_v, a1_v)
    b_v = (b0_v, b1_v)
    o_v = (o0_v, o1_v)
    c = lax.axis_index("c")
    s = lax.axis_index("s")
    wid = c * NS + s
    node_base = s * NPT
    base = wid * EDGES_PER_W

    def load(b, j):
        e = base + j * CHUNK_B
        pltpu.async_copy(eidx_hbm.at[pl.ds(e, CHUNK_B)], sidx_v[b],
                         lsem.at[b])
        pltpu.async_copy(eidx_hbm.at[pl.ds(N_EDGES + e, CHUNK_B)], didx_v[b],
                         lsem.at[b])
        pltpu.async_copy(w_hbm.at[pl.ds(e, CHUNK_B)], w_v[b], lsem.at[b])

    def wait_load(b, j):
        e = base + j * CHUNK_B
        pltpu.make_async_copy(eidx_hbm.at[pl.ds(e, CHUNK_B)], sidx_v[b],
                              lsem.at[b]).wait()
        pltpu.make_async_copy(eidx_hbm.at[pl.ds(N_EDGES + e, CHUNK_B)],
                              didx_v[b], lsem.at[b]).wait()
        pltpu.make_async_copy(w_hbm.at[pl.ds(e, CHUNK_B)], w_v[b],
                              lsem.at[b]).wait()

    def gather(b):
        pltpu.async_copy(norm_src_sh.at[sidx_v[b]], a_v[b], gsem.at[b])
        pltpu.async_copy(norm_dst_sh.at[didx_v[b]], b_v[b], gsem.at[b])

    def wait_gather(b):
        pltpu.make_async_copy(norm_src_sh.at[sidx_v[b]], a_v[b],
                              gsem.at[b]).wait()
        pltpu.make_async_copy(norm_dst_sh.at[didx_v[b]], b_v[b],
                              gsem.at[b]).wait()

    def mul(b):
        def mul_body(i, _):
            for u in range(4):
                sl = pl.ds((i * 4 + u) * L, L)
                o_v[b][sl] = a_v[b][sl] * b_v[b][sl] * w_v[b][sl]
            return _
        lax.fori_loop(0, CHUNK_B // (4 * L), mul_body, None)

    def store(b, j):
        e = base + j * CHUNK_B
        pltpu.async_copy(o_v[b], out_hbm.at[pl.ds(e, CHUNK_B)], osem.at[b])

    def wait_store(b, j):
        e = base + j * CHUNK_B
        pltpu.make_async_copy(o_v[b], out_hbm.at[pl.ds(e, CHUNK_B)],
                              osem.at[b]).wait()

    load(0, 0)
    load(1, 1)

    def build(t, table_sh):
        pltpu.sync_copy(p_hbm.at[pl.ds(t * NPAD + node_base, NPT)], pbuf0_v)
        pltpu.sync_copy(p_hbm.at[pl.ds((2 + t) * NPAD + node_base, NPT)],
                        pbuf1_v)

        def n_body(j, _):
            sl = pl.ds(j * L, L)
            pbuf0_v[sl] = _rsqrt16(pbuf0_v[sl] + pbuf1_v[sl])
            return _
        lax.fori_loop(0, NPT // L, n_body, None)
        pltpu.sync_copy(pbuf0_v, table_sh.at[pl.ds(node_base, NPT)])
    build(0, norm_src_sh)
    build(1, norm_dst_sh)
    plsc.subcore_barrier()

    wait_load(0, 0)
    gather(0)

    def step(b, j, *, ws=True, ng=True, nl=True):
        wait_gather(b)
        if ws:
            wait_store(b, j - 2)
        if ng:
            wait_load(1 - b, j + 1)
            gather(1 - b)
        mul(b)
        store(b, j)
        if nl:
            load(b, j + 2)
        return

    step(0, 0, ws=False)
    step(1, 1, ws=False)

    def ring(g, _):
        for b, off in ((0, 2), (1, 3)):
            step(b, 2 * g + off)
        return _
    lax.fori_loop(0, (ITERS_B - 5) // 2, ring, None)

    step(0, ITERS_B - 3)
    step(1, ITERS_B - 2, nl=False)
    step(0, ITERS_B - 1, ng=False, nl=False)
    wait_store(1, ITERS_B - 2)
    wait_store(0, ITERS_B - 1)


@jax.jit
def kernel(edge_weight, edge_index):
    eidx = edge_index.reshape(2 * N_EDGES)

    mesh = plsc.VectorSubcoreMesh(core_axis_name="c", subcore_axis_name="s")
    npt_f32 = pltpu.VMEM((NPT,), jnp.float32)
    shared_f32 = pltpu.VMEM_SHARED((NPAD,), jnp.float32)
    sem2 = pltpu.SemaphoreType.DMA((2,))

    kern_a = pl.kernel(
        _body_a,
        out_type=jax.ShapeDtypeStruct((NC * 2 * NPAD,), jnp.float32),
        mesh=mesh,
        scratch_types=(
            [pltpu.VMEM((CHUNK_A,), jnp.int32)] * 4
            + [pltpu.VMEM((CHUNK_A,), jnp.float32)] * 2
            + [npt_f32, sem2, sem2, shared_f32, shared_f32]
        ),
    )
    kern_b = pl.kernel(
        _body_b,
        out_type=jax.ShapeDtypeStruct((N_EDGES,), jnp.float32),
        mesh=mesh,
        scratch_types=(
            [pltpu.VMEM((CHUNK_B,), jnp.int32)] * 4
            + [pltpu.VMEM((CHUNK_B,), jnp.float32)] * 8
            + [npt_f32, npt_f32, sem2, sem2, sem2, shared_f32, shared_f32]
        ),
    )
    partials = kern_a(eidx, edge_weight)
    return kern_b(eidx, edge_weight, partials)
